# Initial kernel scaffold; baseline (speedup 1.0000x reference)
import functools

import jax
import jax.numpy as jnp
from jax import lax
from jax.experimental import pallas as pl
from jax.experimental.pallas import tpu as pltpu
from jax.experimental.pallas import tpu_sc as plsc

N = 10000
E = 320000
D_IN = 128
D_HID = 128
D_OUT = 64

NC = 2
NS = 16
NW = NC * NS
EPW = E // NW
CHUNK = 80
NCH = EPW // CHUNK
RPT = N // NS

BN = 2000


def _mesh():
    return plsc.VectorSubcoreMesh(core_axis_name="c", subcore_axis_name="s")


def _sc_degree(row_t, ones, z16):
    @functools.partial(
        pl.kernel,
        out_type=jax.ShapeDtypeStruct((NC, N, 16), jnp.float32),
        mesh=_mesh(),
        scratch_types=[
            pltpu.VMEM((NCH, CHUNK), jnp.int32),
            pltpu.VMEM((CHUNK, 16), jnp.float32),
            pltpu.VMEM_SHARED((N, 16), jnp.float32),
        ],
    )
    def deg_kernel(row_hbm, ones_hbm, z_hbm, out_hbm, idx_v, ones_v, acc):
        core = lax.axis_index("c")
        sid = lax.axis_index("s")
        wid = core * NS + sid
        s0 = sid * RPT
        pltpu.sync_copy(z_hbm.at[pl.ds(s0, RPT)], acc.at[pl.ds(s0, RPT)])
        pltpu.sync_copy(row_hbm.at[wid], idx_v)
        pltpu.sync_copy(ones_hbm, ones_v)
        plsc.subcore_barrier()

        @pl.loop(0, NCH)
        def _(c):
            pltpu.sync_copy(ones_v, acc.at[idx_v.at[c]], add=True)

        plsc.subcore_barrier()
        pltpu.sync_copy(acc.at[pl.ds(s0, RPT)],
                        out_hbm.at[core, pl.ds(s0, RPT)])

    return deg_kernel(row_t, ones, z16)


def _sc_spmm(table, row_t, col_t, zeros, d):
    @functools.partial(
        pl.kernel,
        out_type=jax.ShapeDtypeStruct((NC, N, d), jnp.float32),
        mesh=_mesh(),
        scratch_types=[
            pltpu.VMEM((NCH, CHUNK), jnp.int32),
            pltpu.VMEM((NCH, CHUNK), jnp.int32),
            pltpu.VMEM((CHUNK, d), jnp.float32),
            pltpu.VMEM_SHARED((N, d), jnp.float32),
            pltpu.SemaphoreType.DMA,
        ],
    )
    def spmm_kernel(tab_hbm, row_hbm, col_hbm, z_hbm, out_hbm,
                    ridx_v, cidx_v, buf, acc, sem):
        core = lax.axis_index("c")
        sid = lax.axis_index("s")
        wid = core * NS + sid
        s0 = sid * RPT
        pltpu.sync_copy(z_hbm.at[pl.ds(s0, RPT)], acc.at[pl.ds(s0, RPT)])
        pltpu.sync_copy(row_hbm.at[wid], ridx_v)
        pltpu.sync_copy(col_hbm.at[wid], cidx_v)
        plsc.subcore_barrier()

        @pl.loop(0, NCH)
        def _(c):
            pltpu.async_copy(tab_hbm.at[ridx_v.at[c]], buf, sem).wait()
            pltpu.sync_copy(buf, acc.at[cidx_v.at[c]], add=True)

        plsc.subcore_barrier()
        pltpu.sync_copy(acc.at[pl.ds(s0, RPT)],
                        out_hbm.at[core, pl.ds(s0, RPT)])

    return spmm_kernel(table, row_t, col_t, zeros)


def _dis(dA, dB):
    return lax.rsqrt(dA[:, :1] + dB[:, :1] + 1.0)


def _tc_scale(x, degA, degB):
    def body(x_ref, dA, dB, xs_ref):
        xs_ref[...] = x_ref[...] * _dis(dA[...], dB[...])

    return pl.pallas_call(
        body,
        out_shape=jax.ShapeDtypeStruct((N, D_IN), jnp.float32),
        grid=(N // BN,),
        in_specs=[
            pl.BlockSpec((BN, D_IN), lambda i: (i, 0)),
            pl.BlockSpec((BN, 16), lambda i: (i, 0)),
            pl.BlockSpec((BN, 16), lambda i: (i, 0)),
        ],
        out_specs=pl.BlockSpec((BN, D_IN), lambda i: (i, 0)),
    )(x, degA, degB)


def _tc_layer1(aggrA, aggrB, xs, degA, degB, W1, b1, W2):
    def body(aA, aB, xs_ref, dA, dB, w1, b1r, w2, g_ref):
        dis = _dis(dA[...], dB[...])
        tot = (aA[...] + aB[...] + xs_ref[...]) * dis
        h = lax.dot_general(tot, w1[...], (((1,), (1,)), ((), ())),
                            preferred_element_type=jnp.float32,
                            precision=lax.Precision.HIGHEST)
        h = jnp.maximum(h + b1r[...], 0.0)
        g = lax.dot_general(h, w2[...], (((1,), (1,)), ((), ())),
                            preferred_element_type=jnp.float32,
                            precision=lax.Precision.HIGHEST)
        g_ref[...] = g * dis

    return pl.pallas_call(
        body,
        out_shape=jax.ShapeDtypeStruct((N, D_OUT), jnp.float32),
        grid=(N // BN,),
        in_specs=[
            pl.BlockSpec((BN, D_HID), lambda i: (i, 0)),
            pl.BlockSpec((BN, D_HID), lambda i: (i, 0)),
            pl.BlockSpec((BN, D_IN), lambda i: (i, 0)),
            pl.BlockSpec((BN, 16), lambda i: (i, 0)),
            pl.BlockSpec((BN, 16), lambda i: (i, 0)),
            pl.BlockSpec((D_HID, D_IN), lambda i: (0, 0)),
            pl.BlockSpec((1, D_HID), lambda i: (0, 0)),
            pl.BlockSpec((D_OUT, D_HID), lambda i: (0, 0)),
        ],
        out_specs=pl.BlockSpec((BN, D_OUT), lambda i: (i, 0)),
    )(aggrA, aggrB, xs, degA, degB, W1, b1, W2)


def _tc_out(aggrA, aggrB, g, degA, degB, b2):
    def body(aA, aB, g_ref, dA, dB, b2r, o_ref):
        dis = _dis(dA[...], dB[...])
        z = (aA[...] + aB[...] + g_ref[...]) * dis + b2r[...]
        m = jnp.max(z, axis=1, keepdims=True)
        lse = jnp.log(jnp.sum(jnp.exp(z - m), axis=1, keepdims=True)) + m
        o_ref[...] = z - lse

    return pl.pallas_call(
        body,
        out_shape=jax.ShapeDtypeStruct((N, D_OUT), jnp.float32),
        grid=(N // BN,),
        in_specs=[
            pl.BlockSpec((BN, D_OUT), lambda i: (i, 0)),
            pl.BlockSpec((BN, D_OUT), lambda i: (i, 0)),
            pl.BlockSpec((BN, D_OUT), lambda i: (i, 0)),
            pl.BlockSpec((BN, 16), lambda i: (i, 0)),
            pl.BlockSpec((BN, 16), lambda i: (i, 0)),
            pl.BlockSpec((1, D_OUT), lambda i: (0, 0)),
        ],
        out_specs=pl.BlockSpec((BN, D_OUT), lambda i: (i, 0)),
    )(aggrA, aggrB, g, degA, degB, b2)


def kernel(x, edge_index, W1, b1, W2, b2):
    ei = edge_index.astype(jnp.int32)
    row = ei[0].reshape(NW, NCH, CHUNK)
    col = ei[1].reshape(NW, NCH, CHUNK)
    ones = jnp.ones((CHUNK, 16), jnp.float32)
    z16 = jnp.zeros((N, 16), jnp.float32)
    z128 = jnp.zeros((N, D_IN), jnp.float32)
    z64 = jnp.zeros((N, D_OUT), jnp.float32)

    degp = _sc_degree(row, ones, z16)
    degA, degB = degp[0], degp[1]
    xs = _tc_scale(x, degA, degB)
    ag1 = _sc_spmm(xs, row, col, z128, D_IN)
    g = _tc_layer1(ag1[0], ag1[1], xs, degA, degB,
                   W1, b1.reshape(1, D_HID), W2)
    ag2 = _sc_spmm(g, row, col, z64, D_OUT)
    return _tc_out(ag2[0], ag2[1], g, degA, degB, b2.reshape(1, D_OUT))


# speedup vs baseline: 20.4762x; 20.4762x over previous
import functools

import jax
import jax.numpy as jnp
from jax import lax
from jax.experimental import pallas as pl
from jax.experimental.pallas import tpu as pltpu
from jax.experimental.pallas import tpu_sc as plsc

N = 10000
NP = 10240
E = 320000
D_IN = 128
D_HID = 128
D_OUT = 64

NC = 2
NS = 16
NW = NC * NS
EPW = E // NW
CHUNK = 80
NCH = EPW // CHUNK
RPT = NP // NS

BN = 2048


def _mesh():
    return plsc.VectorSubcoreMesh(core_axis_name="c", subcore_axis_name="s")


_SC_PARAMS = pltpu.CompilerParams(use_tc_tiling_on_sc=False)


def _sc_degree(row_t, ones, z16):
    @functools.partial(
        pl.kernel,
        out_type=jax.ShapeDtypeStruct((NC, NP, 16), jnp.float32),
        mesh=_mesh(),
        scratch_types=[
            pltpu.VMEM((NCH, CHUNK), jnp.int32),
            pltpu.VMEM((CHUNK, 16), jnp.float32),
            pltpu.VMEM_SHARED((NP, 16), jnp.float32),
        ],
        compiler_params=_SC_PARAMS,
    )
    def deg_kernel(row_hbm, ones_hbm, z_hbm, out_hbm, idx_v, ones_v, acc):
        core = lax.axis_index("c")
        sid = lax.axis_index("s")
        wid = core * NS + sid
        s0 = sid * RPT
        pltpu.sync_copy(z_hbm.at[pl.ds(s0, RPT)], acc.at[pl.ds(s0, RPT)])
        pltpu.sync_copy(row_hbm.at[wid], idx_v)
        pltpu.sync_copy(ones_hbm, ones_v)
        plsc.subcore_barrier()

        @pl.loop(0, NCH)
        def _(c):
            pltpu.sync_copy(ones_v, acc.at[idx_v.at[c]], add=True)

        plsc.subcore_barrier()
        pltpu.sync_copy(acc.at[pl.ds(s0, RPT)],
                        out_hbm.at[core, pl.ds(s0, RPT)])

    return deg_kernel(row_t, ones, z16)


def _sc_spmm(table, row_t, col_t, zeros, d):
    @functools.partial(
        pl.kernel,
        out_type=jax.ShapeDtypeStruct((NC, NP, d), jnp.float32),
        mesh=_mesh(),
        scratch_types=[
            pltpu.VMEM((NCH, CHUNK), jnp.int32),
            pltpu.VMEM((NCH, CHUNK), jnp.int32),
            pltpu.VMEM((CHUNK, d), jnp.float32),
            pltpu.VMEM_SHARED((NP, d), jnp.float32),
            pltpu.SemaphoreType.DMA,
        ],
        compiler_params=_SC_PARAMS,
    )
    def spmm_kernel(tab_hbm, row_hbm, col_hbm, z_hbm, out_hbm,
                    ridx_v, cidx_v, buf, acc, sem):
        core = lax.axis_index("c")
        sid = lax.axis_index("s")
        wid = core * NS + sid
        s0 = sid * RPT
        pltpu.sync_copy(z_hbm.at[pl.ds(s0, RPT)], acc.at[pl.ds(s0, RPT)])
        pltpu.sync_copy(row_hbm.at[wid], ridx_v)
        pltpu.sync_copy(col_hbm.at[wid], cidx_v)
        plsc.subcore_barrier()

        @pl.loop(0, NCH)
        def _(c):
            pltpu.async_copy(tab_hbm.at[ridx_v.at[c]], buf, sem).wait()
            pltpu.sync_copy(buf, acc.at[cidx_v.at[c]], add=True)

        plsc.subcore_barrier()
        pltpu.sync_copy(acc.at[pl.ds(s0, RPT)],
                        out_hbm.at[core, pl.ds(s0, RPT)])

    return spmm_kernel(table, row_t, col_t, zeros)


def _dis(dA, dB):
    return lax.rsqrt(dA[:, :1] + dB[:, :1] + 1.0)


def _tc_scale(x, degA, degB):
    def body(x_ref, dA, dB, xs_ref):
        xs_ref[...] = x_ref[...] * _dis(dA[...], dB[...])

    return pl.pallas_call(
        body,
        out_shape=jax.ShapeDtypeStruct((NP, D_IN), jnp.float32),
        grid=(NP // BN,),
        in_specs=[
            pl.BlockSpec((BN, D_IN), lambda i: (i, 0)),
            pl.BlockSpec((BN, 16), lambda i: (i, 0)),
            pl.BlockSpec((BN, 16), lambda i: (i, 0)),
        ],
        out_specs=pl.BlockSpec((BN, D_IN), lambda i: (i, 0)),
    )(x, degA, degB)


def _tc_layer1(aggrA, aggrB, xs, degA, degB, W1, b1, W2):
    def body(aA, aB, xs_ref, dA, dB, w1, b1r, w2, g_ref):
        dis = _dis(dA[...], dB[...])
        tot = (aA[...] + aB[...] + xs_ref[...]) * dis
        h = lax.dot_general(tot, w1[...], (((1,), (1,)), ((), ())),
                            preferred_element_type=jnp.float32,
                            precision=lax.Precision.HIGHEST)
        h = jnp.maximum(h + b1r[...], 0.0)
        g = lax.dot_general(h, w2[...], (((1,), (1,)), ((), ())),
                            preferred_element_type=jnp.float32,
                            precision=lax.Precision.HIGHEST)
        g_ref[...] = g * dis

    return pl.pallas_call(
        body,
        out_shape=jax.ShapeDtypeStruct((NP, D_OUT), jnp.float32),
        grid=(NP // BN,),
        in_specs=[
            pl.BlockSpec((BN, D_HID), lambda i: (i, 0)),
            pl.BlockSpec((BN, D_HID), lambda i: (i, 0)),
            pl.BlockSpec((BN, D_IN), lambda i: (i, 0)),
            pl.BlockSpec((BN, 16), lambda i: (i, 0)),
            pl.BlockSpec((BN, 16), lambda i: (i, 0)),
            pl.BlockSpec((D_HID, D_IN), lambda i: (0, 0)),
            pl.BlockSpec((1, D_HID), lambda i: (0, 0)),
            pl.BlockSpec((D_OUT, D_HID), lambda i: (0, 0)),
        ],
        out_specs=pl.BlockSpec((BN, D_OUT), lambda i: (i, 0)),
    )(aggrA, aggrB, xs, degA, degB, W1, b1, W2)


def _tc_out(aggrA, aggrB, g, degA, degB, b2):
    def body(aA, aB, g_ref, dA, dB, b2r, o_ref):
        dis = _dis(dA[...], dB[...])
        z = (aA[...] + aB[...] + g_ref[...]) * dis + b2r[...]
        m = jnp.max(z, axis=1, keepdims=True)
        lse = jnp.log(jnp.sum(jnp.exp(z - m), axis=1, keepdims=True)) + m
        o_ref[...] = z - lse

    return pl.pallas_call(
        body,
        out_shape=jax.ShapeDtypeStruct((NP, D_OUT), jnp.float32),
        grid=(NP // BN,),
        in_specs=[
            pl.BlockSpec((BN, D_OUT), lambda i: (i, 0)),
            pl.BlockSpec((BN, D_OUT), lambda i: (i, 0)),
            pl.BlockSpec((BN, D_OUT), lambda i: (i, 0)),
            pl.BlockSpec((BN, 16), lambda i: (i, 0)),
            pl.BlockSpec((BN, 16), lambda i: (i, 0)),
            pl.BlockSpec((1, D_OUT), lambda i: (0, 0)),
        ],
        out_specs=pl.BlockSpec((BN, D_OUT), lambda i: (i, 0)),
    )(aggrA, aggrB, g, degA, degB, b2)


def kernel(x, edge_index, W1, b1, W2, b2):
    ei = edge_index.astype(jnp.int32)
    row = ei[0].reshape(NW, NCH, CHUNK)
    col = ei[1].reshape(NW, NCH, CHUNK)
    ones = jnp.ones((CHUNK, 16), jnp.float32)
    z16 = jnp.zeros((NP, 16), jnp.float32)
    z128 = jnp.zeros((NP, D_IN), jnp.float32)
    z64 = jnp.zeros((NP, D_OUT), jnp.float32)
    xp = jnp.pad(x, ((0, NP - N), (0, 0)))

    degp = _sc_degree(row, ones, z16)
    degA, degB = degp[0], degp[1]
    xs = _tc_scale(xp, degA, degB)
    ag1 = _sc_spmm(xs, row, col, z128, D_IN)
    g = _tc_layer1(ag1[0], ag1[1], xs, degA, degB,
                   W1, b1.reshape(1, D_HID), W2)
    ag2 = _sc_spmm(g, row, col, z64, D_OUT)
    out = _tc_out(ag2[0], ag2[1], g, degA, degB, b2.reshape(1, D_OUT))
    return out[:N]


# speedup vs baseline: 31.6907x; 1.5477x over previous
import functools

import jax
import jax.numpy as jnp
from jax import lax
from jax.experimental import pallas as pl
from jax.experimental.pallas import tpu as pltpu
from jax.experimental.pallas import tpu_sc as plsc

N = 10000
NP = 10240
E = 320000
D_IN = 128
D_HID = 128
D_OUT = 64

NC = 2
NS = 16
NW = NC * NS
EPW = E // NW
CHUNK = 80
NCH = EPW // CHUNK
RPT = NP // NS
NB = 4
NBI = 8

BN = 2048


def _mesh():
    return plsc.VectorSubcoreMesh(core_axis_name="c", subcore_axis_name="s")


_SC_PARAMS = pltpu.CompilerParams(use_tc_tiling_on_sc=False)


def _sc_degree(row_t, ones, z16):
    @functools.partial(
        pl.kernel,
        out_type=jax.ShapeDtypeStruct((NC, NP, 8), jnp.float32),
        mesh=_mesh(),
        scratch_types=[
            pltpu.VMEM((NCH, CHUNK), jnp.int32),
            pltpu.VMEM((CHUNK, 8), jnp.float32),
            pltpu.VMEM_SHARED((NP, 8), jnp.float32),
            pltpu.SemaphoreType.DMA,
        ],
        compiler_params=_SC_PARAMS,
    )
    def deg_kernel(row_hbm, ones_hbm, z_hbm, out_hbm, idx_v, ones_v, acc, sem):
        core = lax.axis_index("c")
        sid = lax.axis_index("s")
        wid = core * NS + sid
        s0 = sid * RPT
        pltpu.sync_copy(z_hbm.at[pl.ds(s0, RPT)], acc.at[pl.ds(s0, RPT)])
        pltpu.sync_copy(row_hbm.at[wid], idx_v)
        pltpu.sync_copy(ones_hbm, ones_v)
        plsc.subcore_barrier()

        @pl.loop(0, NCH, step=25)
        def _(c0):
            @pl.loop(0, 25)
            def _(k):
                pltpu.async_copy(ones_v, acc.at[idx_v.at[c0 + k]], sem,
                                 add=True)

            @pl.loop(0, 25)
            def _(k):
                pltpu.make_async_copy(ones_v, acc.at[idx_v.at[c0 + k]],
                                      sem).wait()

        plsc.subcore_barrier()
        pltpu.sync_copy(acc.at[pl.ds(s0, RPT)],
                        out_hbm.at[core, pl.ds(s0, RPT)])

    return deg_kernel(row_t, ones, z16)


def _sc_spmm(table, ec_t, zeros, d):
    @functools.partial(
        pl.kernel,
        out_type=jax.ShapeDtypeStruct((NC, NP, d), jnp.float32),
        mesh=_mesh(),
        scratch_types=[
            pltpu.VMEM_SHARED((NP, d), jnp.float32),
            [pltpu.VMEM((2, CHUNK), jnp.int32)] * NBI,
            [pltpu.VMEM((CHUNK, d), jnp.float32)] * NB,
            [pltpu.SemaphoreType.DMA] * NBI,
            [pltpu.SemaphoreType.DMA] * NB,
            [pltpu.SemaphoreType.DMA] * NB,
        ],
        compiler_params=_SC_PARAMS,
    )
    def spmm_kernel(tab_hbm, ec_hbm, z_hbm, out_hbm,
                    acc, ibufs, bufs, isems, gsems, ssems):
        core = lax.axis_index("c")
        sid = lax.axis_index("s")
        wid = core * NS + sid
        s0 = sid * RPT
        pltpu.sync_copy(z_hbm.at[pl.ds(s0, RPT)], acc.at[pl.ds(s0, RPT)])
        plsc.subcore_barrier()

        def iload(c, j, w=False):
            cp = (pltpu.make_async_copy if w else pltpu.async_copy)(
                ec_hbm.at[wid, c], ibufs[j], isems[j])
            if w:
                cp.wait()

        def gather(c, j, ji, w=False):
            cp = (pltpu.make_async_copy if w else pltpu.async_copy)(
                tab_hbm.at[ibufs[ji].at[0]], bufs[j], gsems[j])
            if w:
                cp.wait()

        def scat(j, ji):
            pltpu.async_copy(bufs[j], acc.at[ibufs[ji].at[1]],
                             ssems[j], add=True)

        def scat_wait(j, ji):
            pltpu.make_async_copy(bufs[j], acc.at[ibufs[ji].at[1]],
                                  ssems[j]).wait()

        def slot(s, m, swait=True, pre=True, post=True):
            gather(s, m % NB, m % NBI, w=True)
            scat(m % NB, m % NBI)
            if swait:
                scat_wait((m - 2) % NB, (m - 2) % NBI)
            if pre:
                iload(s + 2, (m + 2) % NBI, w=True)
                gather(s + 2, (m + 2) % NB, (m + 2) % NBI)
            if post:
                iload(s + 6, (m + 6) % NBI)

        for c in range(6):
            iload(c, c)
        for c in range(2):
            iload(c, c, w=True)
            gather(c, c, c)
        slot(0, 0, swait=False)
        slot(1, 1, swait=False)

        @pl.loop(0, (NCH - 13) // 8)
        def _(i):
            for k in range(8):
                slot(8 * i + 2 + k, 2 + k)

        for s in range(NCH - 11, NCH):
            slot(s, s, pre=(s + 2 < NCH), post=(s + 6 < NCH))
        scat_wait((NCH - 2) % NB, (NCH - 2) % NBI)
        scat_wait((NCH - 1) % NB, (NCH - 1) % NBI)

        plsc.subcore_barrier()
        pltpu.sync_copy(acc.at[pl.ds(s0, RPT)],
                        out_hbm.at[core, pl.ds(s0, RPT)])

    return spmm_kernel(table, ec_t, zeros)


def _dis(dref):
    return lax.rsqrt(dref[0, :, :1] + dref[1, :, :1] + 1.0)


def _deg_spec():
    return pl.BlockSpec((NC, BN, 8), lambda i: (0, i, 0))


def _tc_scale(x, deg):
    def body(x_ref, d_ref, xs_ref):
        xs_ref[...] = x_ref[...] * _dis(d_ref)

    return pl.pallas_call(
        body,
        out_shape=jax.ShapeDtypeStruct((NP, D_IN), jnp.float32),
        grid=(NP // BN,),
        in_specs=[
            pl.BlockSpec((BN, D_IN), lambda i: (i, 0)),
            _deg_spec(),
        ],
        out_specs=pl.BlockSpec((BN, D_IN), lambda i: (i, 0)),
    )(x, deg)


def _tc_layer1(ag, xs, deg, W1, b1, W2):
    def body(a_ref, xs_ref, d_ref, w1, b1r, w2, g_ref):
        dis = _dis(d_ref)
        tot = (a_ref[0] + a_ref[1] + xs_ref[...]) * dis
        h = lax.dot_general(tot, w1[...], (((1,), (1,)), ((), ())),
                            preferred_element_type=jnp.float32,
                            precision=lax.Precision.HIGHEST)
        h = jnp.maximum(h + b1r[...], 0.0)
        g = lax.dot_general(h, w2[...], (((1,), (1,)), ((), ())),
                            preferred_element_type=jnp.float32,
                            precision=lax.Precision.HIGHEST)
        g_ref[...] = g * dis

    return pl.pallas_call(
        body,
        out_shape=jax.ShapeDtypeStruct((NP, D_OUT), jnp.float32),
        grid=(NP // BN,),
        in_specs=[
            pl.BlockSpec((NC, BN, D_HID), lambda i: (0, i, 0)),
            pl.BlockSpec((BN, D_IN), lambda i: (i, 0)),
            _deg_spec(),
            pl.BlockSpec((D_HID, D_IN), lambda i: (0, 0)),
            pl.BlockSpec((1, D_HID), lambda i: (0, 0)),
            pl.BlockSpec((D_OUT, D_HID), lambda i: (0, 0)),
        ],
        out_specs=pl.BlockSpec((BN, D_OUT), lambda i: (i, 0)),
    )(ag, xs, deg, W1, b1, W2)


def _tc_out(ag, g, deg, b2):
    def body(a_ref, g_ref, d_ref, b2r, o_ref):
        dis = _dis(d_ref)
        z = (a_ref[0] + a_ref[1] + g_ref[...]) * dis + b2r[...]
        m = jnp.max(z, axis=1, keepdims=True)
        lse = jnp.log(jnp.sum(jnp.exp(z - m), axis=1, keepdims=True)) + m
        o_ref[...] = z - lse

    return pl.pallas_call(
        body,
        out_shape=jax.ShapeDtypeStruct((NP, D_OUT), jnp.float32),
        grid=(NP // BN,),
        in_specs=[
            pl.BlockSpec((NC, BN, D_OUT), lambda i: (0, i, 0)),
            pl.BlockSpec((BN, D_OUT), lambda i: (i, 0)),
            _deg_spec(),
            pl.BlockSpec((1, D_OUT), lambda i: (0, 0)),
        ],
        out_specs=pl.BlockSpec((BN, D_OUT), lambda i: (i, 0)),
    )(ag, g, deg, b2)


def kernel(x, edge_index, W1, b1, W2, b2):
    ei = edge_index.astype(jnp.int32)
    row = ei[0].reshape(NW, NCH, CHUNK)
    ec = ei.reshape(2, NW, NCH, CHUNK).transpose(1, 2, 0, 3)
    ones = jnp.ones((CHUNK, 8), jnp.float32)
    z16 = jnp.zeros((NP, 8), jnp.float32)
    z128 = jnp.zeros((NP, D_IN), jnp.float32)
    z64 = jnp.zeros((NP, D_OUT), jnp.float32)
    xp = jnp.pad(x, ((0, NP - N), (0, 0)))

    deg = _sc_degree(row, ones, z16)
    xs = _tc_scale(xp, deg)
    ag1 = _sc_spmm(xs, ec, z128, D_IN)
    g = _tc_layer1(ag1, xs, deg, W1, b1.reshape(1, D_HID), W2)
    ag2 = _sc_spmm(g, ec, z64, D_OUT)
    out = _tc_out(ag2, g, deg, b2.reshape(1, D_OUT))
    return out[:N]


# speedup vs baseline: 34.9936x; 1.1042x over previous
import functools

import jax
import jax.numpy as jnp
from jax import lax
from jax.experimental import pallas as pl
from jax.experimental.pallas import tpu as pltpu
from jax.experimental.pallas import tpu_sc as plsc

N = 10000
E = 320000
D_IN = 128
D_HID = 128
D_OUT = 64

NC = 2
NS = 16
NW = NC * NS
EPW = E // NW
CHUNK = 80
NCH = EPW // CHUNK
RPT = 640
RPT_LAST = N - (NS - 1) * RPT
NB = 4
NBI = 8

BN = 2000


def _mesh():
    return plsc.VectorSubcoreMesh(core_axis_name="c", subcore_axis_name="s")


_SC_PARAMS = pltpu.CompilerParams(use_tc_tiling_on_sc=False)


def _stripe_copy(sid, src, dst):
    s0 = sid * RPT

    @pl.when(sid < NS - 1)
    def _():
        pltpu.sync_copy(src.at[pl.ds(s0, RPT)], dst.at[pl.ds(s0, RPT)])

    @pl.when(sid == NS - 1)
    def _():
        pltpu.sync_copy(src.at[pl.ds(s0, RPT_LAST)],
                        dst.at[pl.ds(s0, RPT_LAST)])


def _sc_degree(row_t, ones, z8):
    @functools.partial(
        pl.kernel,
        out_type=jax.ShapeDtypeStruct((NC, N, 8), jnp.float32),
        mesh=_mesh(),
        scratch_types=[
            pltpu.VMEM((NCH, CHUNK), jnp.int32),
            pltpu.VMEM((CHUNK, 8), jnp.float32),
            pltpu.VMEM_SHARED((N, 8), jnp.float32),
            pltpu.SemaphoreType.DMA,
        ],
        compiler_params=_SC_PARAMS,
    )
    def deg_kernel(row_hbm, ones_hbm, z_hbm, out_hbm, idx_v, ones_v, acc, sem):
        core = lax.axis_index("c")
        sid = lax.axis_index("s")
        wid = core * NS + sid
        _stripe_copy(sid, z_hbm, acc)
        pltpu.sync_copy(row_hbm.at[wid], idx_v)
        pltpu.sync_copy(ones_hbm, ones_v)
        plsc.subcore_barrier()

        @pl.loop(0, NCH, step=25)
        def _(c0):
            @pl.loop(0, 25)
            def _(k):
                pltpu.async_copy(ones_v, acc.at[idx_v.at[c0 + k]], sem,
                                 add=True)

            @pl.loop(0, 25)
            def _(k):
                pltpu.make_async_copy(ones_v, acc.at[idx_v.at[c0 + k]],
                                      sem).wait()

        plsc.subcore_barrier()
        _stripe_copy(sid, acc, out_hbm.at[core])

    return deg_kernel(row_t, ones, z8)


def _sc_spmm(table, row_t, col_t, zeros, d):
    @functools.partial(
        pl.kernel,
        out_type=jax.ShapeDtypeStruct((NC, N, d), jnp.float32),
        mesh=_mesh(),
        scratch_types=[
            pltpu.VMEM_SHARED((N, d), jnp.float32),
            [pltpu.VMEM((CHUNK,), jnp.int32)] * NBI,
            [pltpu.VMEM((CHUNK,), jnp.int32)] * NBI,
            [pltpu.VMEM((CHUNK, d), jnp.float32)] * NB,
            [pltpu.SemaphoreType.DMA] * NBI,
            [pltpu.SemaphoreType.DMA] * NBI,
            [pltpu.SemaphoreType.DMA] * NB,
            [pltpu.SemaphoreType.DMA] * NB,
        ],
        compiler_params=_SC_PARAMS,
    )
    def spmm_kernel(tab_hbm, row_hbm, col_hbm, z_hbm, out_hbm,
                    acc, rbufs, cbufs, bufs, rsems, csems, gsems, ssems):
        core = lax.axis_index("c")
        sid = lax.axis_index("s")
        wid = core * NS + sid
        _stripe_copy(sid, z_hbm, acc)
        plsc.subcore_barrier()

        def iload(c, j, w=False):
            if w:
                pltpu.make_async_copy(row_hbm.at[wid, c], rbufs[j],
                                      rsems[j]).wait()
                pltpu.make_async_copy(col_hbm.at[wid, c], cbufs[j],
                                      csems[j]).wait()
            else:
                pltpu.async_copy(row_hbm.at[wid, c], rbufs[j], rsems[j])
                pltpu.async_copy(col_hbm.at[wid, c], cbufs[j], csems[j])

        def gather(c, j, ji, w=False):
            cp = (pltpu.make_async_copy if w else pltpu.async_copy)(
                tab_hbm.at[rbufs[ji]], bufs[j], gsems[j])
            if w:
                cp.wait()

        def scat(j, ji):
            pltpu.async_copy(bufs[j], acc.at[cbufs[ji]], ssems[j], add=True)

        def scat_wait(j, ji):
            pltpu.make_async_copy(bufs[j], acc.at[cbufs[ji]],
                                  ssems[j]).wait()

        def slot(s, m, swait=True, pre=True, post=True):
            gather(s, m % NB, m % NBI, w=True)
            scat(m % NB, m % NBI)
            if swait:
                scat_wait((m - 2) % NB, (m - 2) % NBI)
            if pre:
                iload(s + 2, (m + 2) % NBI, w=True)
                gather(s + 2, (m + 2) % NB, (m + 2) % NBI)
            if post:
                iload(s + 6, (m + 6) % NBI)

        for c in range(6):
            iload(c, c)
        for c in range(2):
            iload(c, c, w=True)
            gather(c, c, c)
        slot(0, 0, swait=False)
        slot(1, 1, swait=False)

        @pl.loop(0, (NCH - 13) // 8)
        def _(i):
            for k in range(8):
                slot(8 * i + 2 + k, 2 + k)

        for s in range(NCH - 11, NCH):
            slot(s, s, pre=(s + 2 < NCH), post=(s + 6 < NCH))
        scat_wait((NCH - 2) % NB, (NCH - 2) % NBI)
        scat_wait((NCH - 1) % NB, (NCH - 1) % NBI)

        plsc.subcore_barrier()
        _stripe_copy(sid, acc, out_hbm.at[core])

    return spmm_kernel(table, row_t, col_t, zeros)


def _dis(dref):
    return lax.rsqrt(dref[0, :, :1] + dref[1, :, :1] + 1.0)


def _deg_spec():
    return pl.BlockSpec((NC, BN, 8), lambda i: (0, i, 0))


def _tc_scale(x, deg):
    def body(x_ref, d_ref, xs_ref):
        xs_ref[...] = x_ref[...] * _dis(d_ref)

    return pl.pallas_call(
        body,
        out_shape=jax.ShapeDtypeStruct((N, D_IN), jnp.float32),
        grid=(N // BN,),
        in_specs=[
            pl.BlockSpec((BN, D_IN), lambda i: (i, 0)),
            _deg_spec(),
        ],
        out_specs=pl.BlockSpec((BN, D_IN), lambda i: (i, 0)),
    )(x, deg)


def _tc_layer1(ag, xs, deg, W1, b1, W2):
    def body(a_ref, xs_ref, d_ref, w1, b1r, w2, g_ref):
        dis = _dis(d_ref)
        tot = (a_ref[0] + a_ref[1] + xs_ref[...]) * dis
        h = lax.dot_general(tot, w1[...], (((1,), (1,)), ((), ())),
                            preferred_element_type=jnp.float32)
        h = jnp.maximum(h + b1r[...], 0.0)
        g = lax.dot_general(h, w2[...], (((1,), (1,)), ((), ())),
                            preferred_element_type=jnp.float32)
        g_ref[...] = g * dis

    return pl.pallas_call(
        body,
        out_shape=jax.ShapeDtypeStruct((N, D_OUT), jnp.float32),
        grid=(N // BN,),
        in_specs=[
            pl.BlockSpec((NC, BN, D_HID), lambda i: (0, i, 0)),
            pl.BlockSpec((BN, D_IN), lambda i: (i, 0)),
            _deg_spec(),
            pl.BlockSpec((D_HID, D_IN), lambda i: (0, 0)),
            pl.BlockSpec((1, D_HID), lambda i: (0, 0)),
            pl.BlockSpec((D_OUT, D_HID), lambda i: (0, 0)),
        ],
        out_specs=pl.BlockSpec((BN, D_OUT), lambda i: (i, 0)),
    )(ag, xs, deg, W1, b1, W2)


def _tc_out(ag, g, deg, b2):
    def body(a_ref, g_ref, d_ref, b2r, o_ref):
        dis = _dis(d_ref)
        z = (a_ref[0] + a_ref[1] + g_ref[...]) * dis + b2r[...]
        m = jnp.max(z, axis=1, keepdims=True)
        lse = jnp.log(jnp.sum(jnp.exp(z - m), axis=1, keepdims=True)) + m
        o_ref[...] = z - lse

    return pl.pallas_call(
        body,
        out_shape=jax.ShapeDtypeStruct((N, D_OUT), jnp.float32),
        grid=(N // BN,),
        in_specs=[
            pl.BlockSpec((NC, BN, D_OUT), lambda i: (0, i, 0)),
            pl.BlockSpec((BN, D_OUT), lambda i: (i, 0)),
            _deg_spec(),
            pl.BlockSpec((1, D_OUT), lambda i: (0, 0)),
        ],
        out_specs=pl.BlockSpec((BN, D_OUT), lambda i: (i, 0)),
    )(ag, g, deg, b2)


def kernel(x, edge_index, W1, b1, W2, b2):
    ei = edge_index.astype(jnp.int32)
    row = ei[0].reshape(NW, NCH, CHUNK)
    col = ei[1].reshape(NW, NCH, CHUNK)
    ones = jnp.ones((CHUNK, 8), jnp.float32)
    z8 = jnp.zeros((N, 8), jnp.float32)
    z128 = jnp.zeros((N, D_IN), jnp.float32)
    z64 = jnp.zeros((N, D_OUT), jnp.float32)

    deg = _sc_degree(row, ones, z8)
    xs = _tc_scale(x, deg)
    ag1 = _sc_spmm(xs, row, col, z128, D_IN)
    g = _tc_layer1(ag1, xs, deg, W1, b1.reshape(1, D_HID), W2)
    ag2 = _sc_spmm(g, row, col, z64, D_OUT)
    return _tc_out(ag2, g, deg, b2.reshape(1, D_OUT))


# speedup vs baseline: 36.7130x; 1.0491x over previous
import functools

import jax
import jax.numpy as jnp
from jax import lax
from jax.experimental import pallas as pl
from jax.experimental.pallas import tpu as pltpu
from jax.experimental.pallas import tpu_sc as plsc

N = 10000
E = 320000
D_IN = 128
D_HID = 128
D_OUT = 64

NC = 2
NS = 16
NW = NC * NS
EPW = E // NW
CHUNK = 128
NCH = EPW // CHUNK
TAIL = EPW - NCH * CHUNK
CHD = 80
NCHD = EPW // CHD
RPT = 640
RPT_LAST = N - (NS - 1) * RPT
NB = 3
NBI = 6

BN = 2000


def _mesh():
    return plsc.VectorSubcoreMesh(core_axis_name="c", subcore_axis_name="s")


_SC_PARAMS = pltpu.CompilerParams(use_tc_tiling_on_sc=False)


def _stripe_copy(sid, src, dst):
    s0 = sid * RPT

    @pl.when(sid < NS - 1)
    def _():
        pltpu.sync_copy(src.at[pl.ds(s0, RPT)], dst.at[pl.ds(s0, RPT)])

    @pl.when(sid == NS - 1)
    def _():
        pltpu.sync_copy(src.at[pl.ds(s0, RPT_LAST)],
                        dst.at[pl.ds(s0, RPT_LAST)])


def _sc_degree(er, ones, z8):
    @functools.partial(
        pl.kernel,
        out_type=jax.ShapeDtypeStruct((NC, N, 8), jnp.float32),
        mesh=_mesh(),
        scratch_types=[
            pltpu.VMEM((CHD, 8), jnp.float32),
            pltpu.VMEM_SHARED((N, 8), jnp.float32),
            [pltpu.VMEM((CHD,), jnp.int32)] * 4,
            [pltpu.SemaphoreType.DMA] * 4,
            [pltpu.SemaphoreType.DMA] * 4,
        ],
        compiler_params=_SC_PARAMS,
    )
    def deg_kernel(er_hbm, ones_hbm, z_hbm, out_hbm, ones_v, acc,
                   ibufs, isems, ssems):
        core = lax.axis_index("c")
        sid = lax.axis_index("s")
        wid = core * NS + sid
        _stripe_copy(sid, z_hbm, acc)
        pltpu.sync_copy(ones_hbm, ones_v)
        plsc.subcore_barrier()

        def iload(c, j, w=False):
            cp = (pltpu.make_async_copy if w else pltpu.async_copy)(
                er_hbm.at[0, wid, pl.ds(c * CHD, CHD)], ibufs[j], isems[j])
            if w:
                cp.wait()

        def scat(j):
            pltpu.async_copy(ones_v, acc.at[ibufs[j]], ssems[j], add=True)

        def scat_wait(j):
            pltpu.make_async_copy(ones_v, acc.at[ibufs[j]],
                                  ssems[j]).wait()

        def slot(s, m, swait=True, post=True):
            iload(s, m % 4, w=True)
            scat(m % 4)
            if swait:
                scat_wait((m - 2) % 4)
            if post:
                iload(s + 2, (m + 2) % 4)

        iload(0, 0)
        iload(1, 1)
        slot(0, 0, swait=False)
        slot(1, 1, swait=False)

        @pl.loop(0, (NCHD - 5) // 4)
        def _(i):
            for k in range(4):
                slot(4 * i + 2 + k, 2 + k)

        for s in range(NCHD - 3, NCHD):
            slot(s, s, post=(s + 2 < NCHD))
        scat_wait((NCHD - 2) % 4)
        scat_wait((NCHD - 1) % 4)

        plsc.subcore_barrier()
        _stripe_copy(sid, acc, out_hbm.at[core])

    return deg_kernel(er, ones, z8)


def _sc_spmm(table, er, zeros, d):
    @functools.partial(
        pl.kernel,
        out_type=jax.ShapeDtypeStruct((NC, N, d), jnp.float32),
        mesh=_mesh(),
        scratch_types=[
            pltpu.VMEM_SHARED((N, d), jnp.float32),
            [pltpu.VMEM((CHUNK,), jnp.int32)] * NBI,
            [pltpu.VMEM((CHUNK,), jnp.int32)] * NBI,
            [pltpu.VMEM((CHUNK, d), jnp.float32)] * NB,
            pltpu.VMEM((TAIL,), jnp.int32),
            pltpu.VMEM((TAIL,), jnp.int32),
            [pltpu.SemaphoreType.DMA] * NBI,
            [pltpu.SemaphoreType.DMA] * NBI,
            [pltpu.SemaphoreType.DMA] * NB,
            [pltpu.SemaphoreType.DMA] * NB,
        ],
        compiler_params=_SC_PARAMS,
    )
    def spmm_kernel(tab_hbm, er_hbm, z_hbm, out_hbm,
                    acc, rbufs, cbufs, bufs, rt_v, ct_v,
                    rsems, csems, gsems, ssems):
        core = lax.axis_index("c")
        sid = lax.axis_index("s")
        wid = core * NS + sid
        _stripe_copy(sid, z_hbm, acc)
        plsc.subcore_barrier()

        def iload(c, j, w=False):
            if w:
                pltpu.make_async_copy(er_hbm.at[0, wid, pl.ds(c * CHUNK, CHUNK)],
                                      rbufs[j], rsems[j]).wait()
                pltpu.make_async_copy(er_hbm.at[1, wid, pl.ds(c * CHUNK, CHUNK)],
                                      cbufs[j], csems[j]).wait()
            else:
                pltpu.async_copy(er_hbm.at[0, wid, pl.ds(c * CHUNK, CHUNK)],
                                 rbufs[j], rsems[j])
                pltpu.async_copy(er_hbm.at[1, wid, pl.ds(c * CHUNK, CHUNK)],
                                 cbufs[j], csems[j])

        def gather(j, ji, w=False):
            cp = (pltpu.make_async_copy if w else pltpu.async_copy)(
                tab_hbm.at[rbufs[ji]], bufs[j], gsems[j])
            if w:
                cp.wait()

        def scat(j, ji):
            pltpu.async_copy(bufs[j], acc.at[cbufs[ji]], ssems[j], add=True)

        def scat_wait(j, ji):
            pltpu.make_async_copy(bufs[j], acc.at[cbufs[ji]],
                                  ssems[j]).wait()

        def slot(s, m, swait=True, pre=True, post=True):
            gather(m % NB, m % NBI, w=True)
            scat(m % NB, m % NBI)
            if swait:
                scat_wait((m - 1) % NB, (m - 1) % NBI)
            if pre:
                iload(s + 2, (m + 2) % NBI, w=True)
                gather((m + 2) % NB, (m + 2) % NBI)
            if post:
                iload(s + 4, (m + 4) % NBI)

        for c in range(4):
            iload(c, c)
        for c in range(2):
            iload(c, c, w=True)
            gather(c, c)
        slot(0, 0, swait=False)

        @pl.loop(0, (NCH - 6) // NBI)
        def _(i):
            for k in range(NBI):
                slot(NBI * i + 1 + k, 1 + k)

        for s in range(NCH - 5, NCH):
            slot(s, s, pre=(s + 2 < NCH), post=(s + 4 < NCH))
        scat_wait((NCH - 1) % NB, (NCH - 1) % NBI)

        pltpu.sync_copy(er_hbm.at[0, wid, pl.ds(NCH * CHUNK, TAIL)], rt_v)
        pltpu.sync_copy(er_hbm.at[1, wid, pl.ds(NCH * CHUNK, TAIL)], ct_v)
        pltpu.sync_copy(tab_hbm.at[rt_v], bufs[0].at[pl.ds(0, TAIL)])
        pltpu.sync_copy(bufs[0].at[pl.ds(0, TAIL)], acc.at[ct_v], add=True)

        plsc.subcore_barrier()
        _stripe_copy(sid, acc, out_hbm.at[core])

    return spmm_kernel(table, er, zeros)


def _dis(dref):
    return lax.rsqrt(dref[0, :, :1] + dref[1, :, :1] + 1.0)


def _deg_spec():
    return pl.BlockSpec((NC, BN, 8), lambda i: (0, i, 0))


def _tc_scale(x, deg):
    def body(x_ref, d_ref, xs_ref):
        xs_ref[...] = x_ref[...] * _dis(d_ref)

    return pl.pallas_call(
        body,
        out_shape=jax.ShapeDtypeStruct((N, D_IN), jnp.float32),
        grid=(N // BN,),
        in_specs=[
            pl.BlockSpec((BN, D_IN), lambda i: (i, 0)),
            _deg_spec(),
        ],
        out_specs=pl.BlockSpec((BN, D_IN), lambda i: (i, 0)),
    )(x, deg)


def _tc_layer1(ag, xs, deg, W1, b1, W2):
    def body(a_ref, xs_ref, d_ref, w1, b1r, w2, g_ref):
        dis = _dis(d_ref)
        tot = (a_ref[0] + a_ref[1] + xs_ref[...]) * dis
        h = lax.dot_general(tot, w1[...], (((1,), (1,)), ((), ())),
                            preferred_element_type=jnp.float32)
        h = jnp.maximum(h + b1r[...], 0.0)
        g = lax.dot_general(h, w2[...], (((1,), (1,)), ((), ())),
                            preferred_element_type=jnp.float32)
        g_ref[...] = g * dis

    return pl.pallas_call(
        body,
        out_shape=jax.ShapeDtypeStruct((N, D_OUT), jnp.float32),
        grid=(N // BN,),
        in_specs=[
            pl.BlockSpec((NC, BN, D_HID), lambda i: (0, i, 0)),
            pl.BlockSpec((BN, D_IN), lambda i: (i, 0)),
            _deg_spec(),
            pl.BlockSpec((D_HID, D_IN), lambda i: (0, 0)),
            pl.BlockSpec((1, D_HID), lambda i: (0, 0)),
            pl.BlockSpec((D_OUT, D_HID), lambda i: (0, 0)),
        ],
        out_specs=pl.BlockSpec((BN, D_OUT), lambda i: (i, 0)),
    )(ag, xs, deg, W1, b1, W2)


def _tc_out(ag, g, deg, b2):
    def body(a_ref, g_ref, d_ref, b2r, o_ref):
        dis = _dis(d_ref)
        z = (a_ref[0] + a_ref[1] + g_ref[...]) * dis + b2r[...]
        m = jnp.max(z, axis=1, keepdims=True)
        lse = jnp.log(jnp.sum(jnp.exp(z - m), axis=1, keepdims=True)) + m
        o_ref[...] = z - lse

    return pl.pallas_call(
        body,
        out_shape=jax.ShapeDtypeStruct((N, D_OUT), jnp.float32),
        grid=(N // BN,),
        in_specs=[
            pl.BlockSpec((NC, BN, D_OUT), lambda i: (0, i, 0)),
            pl.BlockSpec((BN, D_OUT), lambda i: (i, 0)),
            _deg_spec(),
            pl.BlockSpec((1, D_OUT), lambda i: (0, 0)),
        ],
        out_specs=pl.BlockSpec((BN, D_OUT), lambda i: (i, 0)),
    )(ag, g, deg, b2)


def kernel(x, edge_index, W1, b1, W2, b2):
    er = edge_index.astype(jnp.int32).reshape(2, NW, EPW)
    ones = jnp.ones((CHD, 8), jnp.float32)
    z8 = jnp.zeros((N, 8), jnp.float32)
    z128 = jnp.zeros((N, D_IN), jnp.float32)
    z64 = jnp.zeros((N, D_OUT), jnp.float32)

    deg = _sc_degree(er, ones, z8)
    xs = _tc_scale(x, deg)
    ag1 = _sc_spmm(xs, er, z128, D_IN)
    g = _tc_layer1(ag1, xs, deg, W1, b1.reshape(1, D_HID), W2)
    ag2 = _sc_spmm(g, er, z64, D_OUT)
    return _tc_out(ag2, g, deg, b2.reshape(1, D_OUT))
